# baseline (speedup 1.0000x reference)
import functools

import jax
import jax.numpy as jnp
from jax import lax
from jax.experimental import pallas as pl
from jax.experimental.pallas import tpu as pltpu
from jax.experimental.pallas import tpu_sc as plsc

DIM = 1024
MEM_DIM = 128
TABLE = 100000
EPS = 1e-06
N_TOK = 2 * 4096

TN = 1024
GRID = N_TOK // TN


def _sc_gather(table, idx):
    info = plsc.get_sparse_core_info()
    nw = info.num_cores * info.num_subcores
    bpw = N_TOK // nw
    mesh = plsc.VectorSubcoreMesh(core_axis_name="c", subcore_axis_name="s")

    @functools.partial(
        pl.kernel,
        mesh=mesh,
        out_type=jax.ShapeDtypeStruct((N_TOK, MEM_DIM), jnp.float32),
        scratch_types=[
            pltpu.VMEM((bpw,), jnp.int32),
            pltpu.VMEM((bpw, MEM_DIM), jnp.float32),
            pltpu.SemaphoreType.DMA,
        ],
    )
    def gather_kernel(table_hbm, idx_hbm, out_hbm, idx_v, rows_v, sem):
        wid = lax.axis_index("s") * info.num_cores + lax.axis_index("c")
        base = wid * bpw
        pltpu.sync_copy(idx_hbm.at[pl.ds(base, bpw)], idx_v)
        pltpu.async_copy(table_hbm.at[idx_v], rows_v, sem).wait()
        pltpu.sync_copy(rows_v, out_hbm.at[pl.ds(base, bpw)])

    return gather_kernel(table, idx)


def _prep_body(rmsw_ref, qw_ref, kw_ref, vw_ref, w_ref, vwt_ref):
    w_ref[...] = lax.dot_general(
        qw_ref[...], kw_ref[...], (((0,), (0,)), ((), ())),
        preferred_element_type=jnp.float32) * rmsw_ref[...].T
    vwt_ref[...] = vw_ref[...].T


def _prep_call(rms_w2, q_w, k_w, v_w):
    return pl.pallas_call(
        _prep_body,
        out_shape=(jax.ShapeDtypeStruct((DIM, MEM_DIM), jnp.float32),
                   jax.ShapeDtypeStruct((MEM_DIM, DIM), jnp.float32)),
    )(rms_w2, q_w, k_w, v_w)


CHUNK = 4
CROWS = TN // CHUNK


def _x_copy(x_hbm, x_v, sems, step, slot, c):
    return pltpu.make_async_copy(
        x_hbm.at[pl.ds(step * TN + c * CROWS, CROWS)],
        x_v.at[slot, pl.ds(c * CROWS, CROWS)],
        sems.at[slot, c])


def _out_copy(out_hbm, out_v, sems, step, slot, c):
    return pltpu.make_async_copy(
        out_v.at[slot, pl.ds(c * CROWS, CROWS)],
        out_hbm.at[pl.ds(step * TN + c * CROWS, CROWS)],
        sems.at[slot, c])


def _tc_body(ids_ref, mem_ref, w_ref, vwt_ref, x_hbm, out_hbm,
             x_v, out_v, in_sems, out_sems):
    i = pl.program_id(0)
    slot = lax.rem(i, 2)
    nslot = lax.rem(i + 1, 2)

    @pl.when(i == 0)
    def _():
        for c in range(CHUNK):
            _x_copy(x_hbm, x_v, in_sems, i, slot, c).start()

    @pl.when(i + 1 < GRID)
    def _():
        for c in range(CHUNK):
            _x_copy(x_hbm, x_v, in_sems, i + 1, nslot, c).start()

    for c in range(CHUNK):
        _x_copy(x_hbm, x_v, in_sems, i, slot, c).wait()

    @pl.when(i >= 2)
    def _():
        for c in range(CHUNK):
            _out_copy(out_hbm, out_v, out_sems, i - 2, slot, c).wait()

    x = x_v[slot]
    var = jnp.mean(x * x, axis=-1, keepdims=True)
    r = jnp.dot(x, w_ref[...], preferred_element_type=jnp.float32)
    mem = mem_ref[...]
    s = jnp.sum(r * mem, axis=-1, keepdims=True)
    logit = s * lax.rsqrt(var + EPS) * (1.0 / 32.0)
    gate = jax.nn.sigmoid(logit)
    gate = gate * (ids_ref[...] != 0).astype(jnp.float32)
    v = jnp.dot(mem, vwt_ref[...], preferred_element_type=jnp.float32)
    out_v[slot] = gate * v

    for c in range(CHUNK):
        _out_copy(out_hbm, out_v, out_sems, i, slot, c).start()

    @pl.when(i == GRID - 1)
    def _():
        for c in range(CHUNK):
            _out_copy(out_hbm, out_v, out_sems, i - 1, nslot, c).wait()
        for c in range(CHUNK):
            _out_copy(out_hbm, out_v, out_sems, i, slot, c).wait()


def _tc_call(ids_col, x2, mem, w, vwt):
    return pl.pallas_call(
        _tc_body,
        grid=(GRID,),
        in_specs=[
            pl.BlockSpec((TN, 1), lambda i: (i, 0)),
            pl.BlockSpec((TN, MEM_DIM), lambda i: (i, 0)),
            pl.BlockSpec((DIM, MEM_DIM), lambda i: (0, 0)),
            pl.BlockSpec((MEM_DIM, DIM), lambda i: (0, 0)),
            pl.BlockSpec(memory_space=pl.ANY),
        ],
        out_specs=pl.BlockSpec(memory_space=pl.ANY),
        out_shape=jax.ShapeDtypeStruct((N_TOK, DIM), jnp.float32),
        scratch_shapes=[
            pltpu.VMEM((2, TN, DIM), jnp.float32),
            pltpu.VMEM((2, TN, DIM), jnp.float32),
            pltpu.SemaphoreType.DMA((2, CHUNK)),
            pltpu.SemaphoreType.DMA((2, CHUNK)),
        ],
    )(ids_col, mem, w, vwt, x2)


def kernel(x, bigram_ids, embed, k_w, v_w, q_w, rms_w):
    ids_flat = bigram_ids.reshape(N_TOK).astype(jnp.int32)
    mem = _sc_gather(embed, ids_flat)
    w, vwt = _prep_call(rms_w.reshape(1, DIM), q_w, k_w, v_w)
    out = _tc_call(ids_flat.reshape(N_TOK, 1), x.reshape(N_TOK, DIM), mem, w,
                   vwt)
    return out.reshape(x.shape)


# speedup vs baseline: 1.0685x; 1.0685x over previous
import functools

import jax
import jax.numpy as jnp
from jax import lax
from jax.experimental import pallas as pl
from jax.experimental.pallas import tpu as pltpu
from jax.experimental.pallas import tpu_sc as plsc

DIM = 1024
MEM_DIM = 128
TABLE = 100000
EPS = 1e-06
B_SZ = 2
S_SZ = 4096
N_TOK = B_SZ * S_SZ

TN = 1024
GRID = N_TOK // TN
CHUNK = 4
CROWS = TN // CHUNK


def _sc_gather(table, idx2d):
    info = plsc.get_sparse_core_info()
    nw = info.num_cores * info.num_subcores
    bpw = N_TOK // nw
    mesh = plsc.VectorSubcoreMesh(core_axis_name="c", subcore_axis_name="s")

    @functools.partial(
        pl.kernel,
        mesh=mesh,
        out_type=jax.ShapeDtypeStruct((N_TOK, MEM_DIM), jnp.float32),
        scratch_types=[
            pltpu.VMEM((bpw,), jnp.int32),
            pltpu.VMEM((bpw, MEM_DIM), jnp.float32),
            pltpu.SemaphoreType.DMA,
        ],
    )
    def gather_kernel(table_hbm, idx_hbm, out_hbm, idx_v, rows_v, sem):
        wid = lax.axis_index("s") * info.num_cores + lax.axis_index("c")
        base = wid * bpw
        row = base // S_SZ
        col = base - row * S_SZ
        pltpu.sync_copy(idx_hbm.at[row, pl.ds(col, bpw)], idx_v)
        pltpu.async_copy(table_hbm.at[idx_v], rows_v, sem).wait()
        pltpu.sync_copy(rows_v, out_hbm.at[pl.ds(base, bpw)])

    return gather_kernel(table, idx2d)


def _hbm_copy(hbm, vmem, sems, step, slot, c):
    return pltpu.make_async_copy(
        hbm.at[pl.ds(step * TN + c * CROWS, CROWS)],
        vmem.at[slot, pl.ds(c * CROWS, CROWS)],
        sems.at[slot, c])


def _a_body(rmsw_ref, kw_ref, qw_hbm, x_hbm, r_ref,
            x_v, w_scr, qw_v, in_sems, qw_sems):
    i = pl.program_id(0)
    slot = lax.rem(i, 2)
    nslot = lax.rem(i + 1, 2)

    @pl.when(i == 0)
    def _():
        for c in range(CHUNK):
            pltpu.make_async_copy(
                qw_hbm.at[pl.ds(c * (DIM // CHUNK), DIM // CHUNK)],
                qw_v.at[pl.ds(c * (DIM // CHUNK), DIM // CHUNK)],
                qw_sems.at[c]).start()
        for c in range(CHUNK):
            _hbm_copy(x_hbm, x_v, in_sems, i, slot, c).start()
        for c in range(CHUNK):
            pltpu.make_async_copy(
                qw_hbm.at[pl.ds(c * (DIM // CHUNK), DIM // CHUNK)],
                qw_v.at[pl.ds(c * (DIM // CHUNK), DIM // CHUNK)],
                qw_sems.at[c]).wait()
        w_scr[...] = lax.dot_general(
            qw_v[...], kw_ref[...], (((0,), (0,)), ((), ())),
            preferred_element_type=jnp.float32) * rmsw_ref[...].T

    @pl.when(i + 1 < GRID)
    def _():
        for c in range(CHUNK):
            _hbm_copy(x_hbm, x_v, in_sems, i + 1, nslot, c).start()

    for c in range(CHUNK):
        _hbm_copy(x_hbm, x_v, in_sems, i, slot, c).wait()

    x = x_v[slot]
    var = jnp.mean(x * x, axis=-1, keepdims=True)
    r = jnp.dot(x, w_scr[...], preferred_element_type=jnp.float32)
    r_ref[...] = r * (lax.rsqrt(var + EPS) * (1.0 / 32.0))


def _a_call(rms_w2, k_w, q_w, x2):
    return pl.pallas_call(
        _a_body,
        grid=(GRID,),
        in_specs=[
            pl.BlockSpec((1, DIM), lambda i: (0, 0)),
            pl.BlockSpec((DIM, MEM_DIM), lambda i: (0, 0)),
            pl.BlockSpec(memory_space=pl.ANY),
            pl.BlockSpec(memory_space=pl.ANY),
        ],
        out_specs=pl.BlockSpec((TN, MEM_DIM), lambda i: (i, 0)),
        out_shape=jax.ShapeDtypeStruct((N_TOK, MEM_DIM), jnp.float32),
        scratch_shapes=[
            pltpu.VMEM((2, TN, DIM), jnp.float32),
            pltpu.VMEM((DIM, MEM_DIM), jnp.float32),
            pltpu.VMEM((DIM, DIM), jnp.float32),
            pltpu.SemaphoreType.DMA((2, CHUNK)),
            pltpu.SemaphoreType.DMA((CHUNK,)),
        ],
    )(rms_w2, k_w, q_w, x2)


def _b_body(ids_ref, r_ref, mem_ref, vw_ref, out_hbm,
            out_v, vwt_scr, out_sems):
    i = pl.program_id(0)
    slot = lax.rem(i, 2)
    nslot = lax.rem(i + 1, 2)

    @pl.when(i == 0)
    def _():
        vwt_scr[...] = vw_ref[...].T

    @pl.when(i >= 2)
    def _():
        for c in range(CHUNK):
            _hbm_copy(out_hbm, out_v, out_sems, i - 2, slot, c).wait()

    mem = mem_ref[...]
    s = jnp.sum(r_ref[...] * mem, axis=-1, keepdims=True)
    gate = jax.nn.sigmoid(s)
    mask = (ids_ref[0] != 0).astype(jnp.float32).T
    gate = gate * mask
    v = jnp.dot(mem, vwt_scr[...], preferred_element_type=jnp.float32)
    out_v[slot] = gate * v

    for c in range(CHUNK):
        _hbm_copy(out_hbm, out_v, out_sems, i, slot, c).start()

    @pl.when(i == GRID - 1)
    def _():
        for c in range(CHUNK):
            _hbm_copy(out_hbm, out_v, out_sems, i - 1, nslot, c).wait()
        for c in range(CHUNK):
            _hbm_copy(out_hbm, out_v, out_sems, i, slot, c).wait()


def _b_call(ids3d, r, mem, v_w):
    return pl.pallas_call(
        _b_body,
        grid=(GRID,),
        in_specs=[
            pl.BlockSpec((1, 1, TN), lambda i: (i, 0, 0)),
            pl.BlockSpec((TN, MEM_DIM), lambda i: (i, 0)),
            pl.BlockSpec((TN, MEM_DIM), lambda i: (i, 0)),
            pl.BlockSpec((DIM, MEM_DIM), lambda i: (0, 0)),
        ],
        out_specs=pl.BlockSpec(memory_space=pl.ANY),
        out_shape=jax.ShapeDtypeStruct((N_TOK, DIM), jnp.float32),
        scratch_shapes=[
            pltpu.VMEM((2, TN, DIM), jnp.float32),
            pltpu.VMEM((MEM_DIM, DIM), jnp.float32),
            pltpu.SemaphoreType.DMA((2, CHUNK)),
        ],
    )(ids3d, r, mem, v_w)


def kernel(x, bigram_ids, embed, k_w, v_w, q_w, rms_w):
    ids2d = bigram_ids.astype(jnp.int32)
    mem = _sc_gather(embed, ids2d)
    r = _a_call(rms_w.reshape(1, DIM), k_w, q_w, x.reshape(N_TOK, DIM))
    out = _b_call(ids2d.reshape(GRID, 1, TN), r, mem, v_w)
    return out.reshape(x.shape)
